# baseline (speedup 1.0000x reference)
import functools

import jax
import jax.numpy as jnp
from jax import lax
from jax.experimental import pallas as pl
from jax.experimental.pallas import tpu as pltpu
from jax.experimental.pallas import tpu_sc as plsc

B = 4096
D = 64
V = 100000
S = 4096
N_IDS = B + S
BROWS = (V + 127) // 128

_NC = 2
_NS = 16
_NW = _NC * _NS
_PER_W = N_IDS // _NW
_CHUNK = 128
_NCHUNK = _PER_W // _CHUNK


_HW = _NW // 2


def _sc_gather_body(idx_hbm, w_hbm, b_hbm, out_tw, out_tb, out_sw, out_sb,
                    idx_v, wrows_v, bval_v, sem):
    wid = lax.axis_index("s") * _NC + lax.axis_index("c")
    pltpu.sync_copy(idx_hbm.at[wid], idx_v)
    copies = []
    for j in range(_NCHUNK):
        copies.append(pltpu.async_copy(w_hbm.at[idx_v.at[j]],
                                       wrows_v.at[j], sem))
        copies.append(pltpu.async_copy(b_hbm.at[idx_v.at[j]],
                                       bval_v.at[j], sem))
    for c in copies:
        c.wait()

    @pl.when(wid < _HW)
    def _true_side():
        base = wid * _PER_W
        for j in range(_NCHUNK):
            pltpu.sync_copy(wrows_v.at[j],
                            out_tw.at[pl.ds(base + j * _CHUNK, _CHUNK)])
            pltpu.sync_copy(bval_v.at[j],
                            out_tb.at[pl.ds(base + j * _CHUNK, _CHUNK)])

    @pl.when(wid >= _HW)
    def _sampled_side():
        base = (wid - _HW) * _PER_W
        for j in range(_NCHUNK):
            pltpu.sync_copy(wrows_v.at[j],
                            out_sw.at[pl.ds(base + j * _CHUNK, _CHUNK)])
            pltpu.sync_copy(bval_v.at[j],
                            out_sb.at[pl.ds(base + j * _CHUNK, _CHUNK)])


@jax.jit
def _sc_gather(idx, weights, biases):
    mesh = plsc.VectorSubcoreMesh(core_axis_name="c", subcore_axis_name="s")
    return pl.kernel(
        _sc_gather_body,
        out_type=(
            jax.ShapeDtypeStruct((B, D), jnp.bfloat16),
            jax.ShapeDtypeStruct((B,), jnp.float32),
            jax.ShapeDtypeStruct((S, D), jnp.bfloat16),
            jax.ShapeDtypeStruct((S,), jnp.float32),
        ),
        mesh=mesh,
        compiler_params=pltpu.CompilerParams(use_tc_tiling_on_sc=False),
        scratch_types=[
            pltpu.VMEM((_NCHUNK, _CHUNK), jnp.int32),
            pltpu.VMEM((_NCHUNK, _CHUNK, D), jnp.bfloat16),
            pltpu.VMEM((_NCHUNK, _CHUNK), jnp.float32),
            pltpu.SemaphoreType.DMA,
        ],
    )(idx, weights, biases)


_TB = 512
_GRID = B // _TB
_SCALE = 1.0 / (B * (S + 1))
_EPS = 1e-12


def _tc_body(x_ref, xf_ref, twr_ref, tb_ref, tec_ref,
             swr_ref, sb_ref, sec_ref, out_ref, rhs_ref):
    i = pl.program_id(0)

    @pl.when(i == 0)
    def _prep():
        rhs_ref[:, 0:D] = swr_ref[...]
        bcol = sb_ref[...] - jnp.log(sec_ref[...])
        lane64 = lax.broadcasted_iota(jnp.int32, (S, 64), 1)
        rhs_ref[:, 64:128] = jnp.where(lane64 == 0, bcol,
                                       0.0).astype(jnp.bfloat16)
        txw = jnp.sum(xf_ref[...] * twr_ref[...].astype(jnp.float32),
                      axis=1, keepdims=True)
        tl = txw + tb_ref[...] - jnp.log(tec_ref[...])
        pt = jax.nn.sigmoid(tl)
        tsum = jnp.sum(-jnp.log(jnp.clip(pt, _EPS, 1.0)))
        out_ref[0, 0] = tsum * _SCALE

    x = x_ref[...]
    xa = jnp.concatenate(
        [x, jnp.ones((_TB, 64), jnp.float32)],
        axis=1).astype(jnp.bfloat16)
    logits = lax.dot_general(
        xa, rhs_ref[...], (((1,), (1,)), ((), ())),
        preferred_element_type=jnp.float32)
    p = jax.nn.sigmoid(logits)
    part = jnp.sum(-jnp.log(jnp.clip(1.0 - p, _EPS, 1.0)))
    out_ref[0, 0] += part * _SCALE


@functools.partial(jax.jit, static_argnames=("interpret",))
def _tc_loss(inputs, twr, tb, tec, swr, sb, sec, interpret=False):
    out = pl.pallas_call(
        _tc_body,
        grid=(_GRID,),
        in_specs=[
            pl.BlockSpec((_TB, D), lambda i: (i, 0)),
            pl.BlockSpec((B, D), lambda i: (0, 0)),
            pl.BlockSpec((B, D), lambda i: (0, 0)),
            pl.BlockSpec((B, 1), lambda i: (0, 0)),
            pl.BlockSpec((B, 1), lambda i: (0, 0)),
            pl.BlockSpec((S, D), lambda i: (0, 0)),
            pl.BlockSpec((S, 1), lambda i: (0, 0)),
            pl.BlockSpec((S, 1), lambda i: (0, 0)),
        ],
        out_specs=pl.BlockSpec(memory_space=pltpu.SMEM),
        out_shape=jax.ShapeDtypeStruct((1, 1), jnp.float32),
        scratch_shapes=[pltpu.VMEM((S, 128), jnp.bfloat16)],
        interpret=interpret,
    )(inputs, inputs, twr, tb, tec, swr, sb, sec)
    return out[0, 0]


def kernel(inputs, labels, weights, biases, sampled_candidates,
           true_expected_count, sampled_expected_count):
    ids = jnp.concatenate(
        [labels.reshape(-1).astype(jnp.int32),
         sampled_candidates.astype(jnp.int32)], axis=0)
    ids3 = ids.reshape(_NW, _NCHUNK, _CHUNK)
    tw, tb, sw, sb = _sc_gather(ids3, weights.astype(jnp.bfloat16), biases)
    return _tc_loss(inputs,
                    tw, tb.reshape(B, 1),
                    true_expected_count,
                    sw, sb.reshape(S, 1),
                    sampled_expected_count.reshape(S, 1))


# speedup vs baseline: 1.2305x; 1.2305x over previous
import functools

import jax
import jax.numpy as jnp
from jax import lax
from jax.experimental import pallas as pl
from jax.experimental.pallas import tpu as pltpu
from jax.experimental.pallas import tpu_sc as plsc

B = 4096
D = 64
V = 100000
S = 4096
N_IDS = B + S
BROWS = (V + 127) // 128

_NC = 2
_NS = 16
_NW = _NC * _NS
_PER_W = N_IDS // _NW
_CHUNK = 128
_NCHUNK = _PER_W // _CHUNK


_HW = _NW // 2


def _sc_gather_body(idx_hbm, w_hbm, b_hbm, out_tw, out_tb, out_sw, out_sb,
                    idx_v, wrows_v, bval_v, sem):
    wid = lax.axis_index("s") * _NC + lax.axis_index("c")
    pltpu.sync_copy(idx_hbm.at[wid], idx_v)
    copies = []
    for j in range(_NCHUNK):
        copies.append(pltpu.async_copy(w_hbm.at[idx_v.at[j]],
                                       wrows_v.at[j], sem))
        copies.append(pltpu.async_copy(b_hbm.at[idx_v.at[j]],
                                       bval_v.at[j], sem))
    for c in copies:
        c.wait()

    @pl.when(wid < _HW)
    def _true_side():
        base = wid * _PER_W
        for j in range(_NCHUNK):
            pltpu.sync_copy(wrows_v.at[j],
                            out_tw.at[pl.ds(base + j * _CHUNK, _CHUNK)])
            pltpu.sync_copy(bval_v.at[j],
                            out_tb.at[pl.ds(base + j * _CHUNK, _CHUNK)])

    @pl.when(wid >= _HW)
    def _sampled_side():
        base = (wid - _HW) * _PER_W
        for j in range(_NCHUNK):
            pltpu.sync_copy(wrows_v.at[j],
                            out_sw.at[pl.ds(base + j * _CHUNK, _CHUNK)])
            pltpu.sync_copy(bval_v.at[j],
                            out_sb.at[pl.ds(base + j * _CHUNK, _CHUNK)])


@jax.jit
def _sc_gather(idx, weights, biases):
    mesh = plsc.VectorSubcoreMesh(core_axis_name="c", subcore_axis_name="s")
    return pl.kernel(
        _sc_gather_body,
        out_type=(
            jax.ShapeDtypeStruct((B, D), jnp.float32),
            jax.ShapeDtypeStruct((B,), jnp.float32),
            jax.ShapeDtypeStruct((S, D), jnp.float32),
            jax.ShapeDtypeStruct((S,), jnp.float32),
        ),
        mesh=mesh,
        compiler_params=pltpu.CompilerParams(use_tc_tiling_on_sc=False),
        scratch_types=[
            pltpu.VMEM((_NCHUNK, _CHUNK), jnp.int32),
            pltpu.VMEM((_NCHUNK, _CHUNK, D), jnp.float32),
            pltpu.VMEM((_NCHUNK, _CHUNK), jnp.float32),
            pltpu.SemaphoreType.DMA,
        ],
    )(idx, weights, biases)


_TB = 512
_GRID = B // _TB
_SCALE = 1.0 / (B * (S + 1))
_EPS = 1e-12


def _tc_body(x_ref, xf_ref, twr_ref, tb_ref, tec_ref,
             swr_ref, sb_ref, sec_ref, out_ref, rhs_ref):
    i = pl.program_id(0)

    @pl.when(i == 0)
    def _prep():
        rhs_ref[:, 0:D] = swr_ref[...].astype(jnp.bfloat16)
        bcol = sb_ref[...] - jnp.log(sec_ref[...])
        lane64 = lax.broadcasted_iota(jnp.int32, (S, 64), 1)
        rhs_ref[:, 64:128] = jnp.where(lane64 == 0, bcol,
                                       0.0).astype(jnp.bfloat16)
        txw = jnp.sum(xf_ref[...] * twr_ref[...], axis=1, keepdims=True)
        tl = txw + tb_ref[...] - jnp.log(tec_ref[...])
        pt = jax.nn.sigmoid(tl)
        tsum = jnp.sum(-jnp.log(jnp.clip(pt, _EPS, 1.0)))
        out_ref[0, 0] = tsum * _SCALE

    x = x_ref[...]
    xa = jnp.concatenate(
        [x, jnp.ones((_TB, 64), jnp.float32)],
        axis=1).astype(jnp.bfloat16)
    logits = lax.dot_general(
        xa, rhs_ref[...], (((1,), (1,)), ((), ())),
        preferred_element_type=jnp.float32)
    sp = (jnp.maximum(logits, 0.0)
          + jnp.log(1.0 + jnp.exp(-jnp.abs(logits))))
    part = jnp.sum(jnp.where(logits >= 17.3286795, 27.631021115928547, sp))
    out_ref[0, 0] += part * _SCALE


@functools.partial(jax.jit, static_argnames=("interpret",))
def _tc_loss(inputs, twr, tb, tec, swr, sb, sec, interpret=False):
    out = pl.pallas_call(
        _tc_body,
        grid=(_GRID,),
        in_specs=[
            pl.BlockSpec((_TB, D), lambda i: (i, 0)),
            pl.BlockSpec((B, D), lambda i: (0, 0)),
            pl.BlockSpec((B, D), lambda i: (0, 0)),
            pl.BlockSpec((B, 1), lambda i: (0, 0)),
            pl.BlockSpec((B, 1), lambda i: (0, 0)),
            pl.BlockSpec((S, D), lambda i: (0, 0)),
            pl.BlockSpec((S, 1), lambda i: (0, 0)),
            pl.BlockSpec((S, 1), lambda i: (0, 0)),
        ],
        out_specs=pl.BlockSpec(memory_space=pltpu.SMEM),
        out_shape=jax.ShapeDtypeStruct((1, 1), jnp.float32),
        scratch_shapes=[pltpu.VMEM((S, 128), jnp.bfloat16)],
        interpret=interpret,
    )(inputs, inputs, twr, tb, tec, swr, sb, sec)
    return out[0, 0]


def kernel(inputs, labels, weights, biases, sampled_candidates,
           true_expected_count, sampled_expected_count):
    ids = jnp.concatenate(
        [labels.reshape(-1).astype(jnp.int32),
         sampled_candidates.astype(jnp.int32)], axis=0)
    ids3 = ids.reshape(_NW, _NCHUNK, _CHUNK)
    tw, tb, sw, sb = _sc_gather(ids3, weights, biases)
    return _tc_loss(inputs,
                    tw, tb.reshape(B, 1),
                    true_expected_count,
                    sw, sb.reshape(S, 1),
                    sampled_expected_count.reshape(S, 1))


# speedup vs baseline: 1.2316x; 1.0009x over previous
import functools

import jax
import jax.numpy as jnp
from jax import lax
from jax.experimental import pallas as pl
from jax.experimental.pallas import tpu as pltpu
from jax.experimental.pallas import tpu_sc as plsc

B = 4096
D = 64
V = 100000
S = 4096
N_IDS = B + S
BROWS = (V + 127) // 128

_NC = 2
_NS = 16
_NW = _NC * _NS
_PER_W = N_IDS // _NW
_CHUNK = 128
_NCHUNK = _PER_W // _CHUNK


_HW = _NW // 2


def _sc_gather_body(idx_hbm, w_hbm, b_hbm, out_tw, out_tb, out_sw, out_sb,
                    idx_v, wrows_v, bval_v, sem):
    wid = lax.axis_index("s") * _NC + lax.axis_index("c")
    pltpu.sync_copy(idx_hbm.at[wid], idx_v)
    copies = []
    for j in range(_NCHUNK):
        copies.append(pltpu.async_copy(w_hbm.at[idx_v.at[j]],
                                       wrows_v.at[j], sem))
        copies.append(pltpu.async_copy(b_hbm.at[idx_v.at[j]],
                                       bval_v.at[j], sem))
    for c in copies:
        c.wait()

    @pl.when(wid < _HW)
    def _true_side():
        base = wid * _PER_W
        for j in range(_NCHUNK):
            pltpu.sync_copy(wrows_v.at[j],
                            out_tw.at[pl.ds(base + j * _CHUNK, _CHUNK)])
            pltpu.sync_copy(bval_v.at[j],
                            out_tb.at[pl.ds(base + j * _CHUNK, _CHUNK)])

    @pl.when(wid >= _HW)
    def _sampled_side():
        base = (wid - _HW) * _PER_W
        for j in range(_NCHUNK):
            pltpu.sync_copy(wrows_v.at[j],
                            out_sw.at[pl.ds(base + j * _CHUNK, _CHUNK)])
            pltpu.sync_copy(bval_v.at[j],
                            out_sb.at[pl.ds(base + j * _CHUNK, _CHUNK)])


@jax.jit
def _sc_gather(idx, weights, biases):
    mesh = plsc.VectorSubcoreMesh(core_axis_name="c", subcore_axis_name="s")
    return pl.kernel(
        _sc_gather_body,
        out_type=(
            jax.ShapeDtypeStruct((B, D), jnp.float32),
            jax.ShapeDtypeStruct((B,), jnp.float32),
            jax.ShapeDtypeStruct((S, D), jnp.float32),
            jax.ShapeDtypeStruct((S,), jnp.float32),
        ),
        mesh=mesh,
        compiler_params=pltpu.CompilerParams(use_tc_tiling_on_sc=False),
        scratch_types=[
            pltpu.VMEM((_NCHUNK, _CHUNK), jnp.int32),
            pltpu.VMEM((_NCHUNK, _CHUNK, D), jnp.float32),
            pltpu.VMEM((_NCHUNK, _CHUNK), jnp.float32),
            pltpu.SemaphoreType.DMA,
        ],
    )(idx, weights, biases)


_TB = 1024
_GRID = B // _TB
_SCALE = 1.0 / (B * (S + 1))
_EPS = 1e-12


def _tc_body(x_ref, xf_ref, twr_ref, tb_ref, tec_ref,
             swr_ref, sb_ref, sec_ref, out_ref, rhs_ref):
    i = pl.program_id(0)

    @pl.when(i == 0)
    def _prep():
        rhs_ref[:, 0:D] = swr_ref[...].astype(jnp.bfloat16)
        bcol = sb_ref[...] - jnp.log(sec_ref[...])
        lane64 = lax.broadcasted_iota(jnp.int32, (S, 64), 1)
        rhs_ref[:, 64:128] = jnp.where(lane64 == 0, bcol,
                                       0.0).astype(jnp.bfloat16)
        txw = jnp.sum(xf_ref[...] * twr_ref[...], axis=1, keepdims=True)
        tl = txw + tb_ref[...] - jnp.log(tec_ref[...])
        pt = jax.nn.sigmoid(tl)
        tsum = jnp.sum(-jnp.log(jnp.clip(pt, _EPS, 1.0)))
        out_ref[0, 0] = tsum * _SCALE

    x = x_ref[...]
    xa = jnp.concatenate(
        [x, jnp.ones((_TB, 64), jnp.float32)],
        axis=1).astype(jnp.bfloat16)
    logits = lax.dot_general(
        xa, rhs_ref[...], (((1,), (1,)), ((), ())),
        preferred_element_type=jnp.float32)
    p = jax.nn.sigmoid(logits)
    part = jnp.sum(-jnp.log(jnp.clip(1.0 - p, _EPS, 1.0)))
    out_ref[0, 0] += part * _SCALE


@functools.partial(jax.jit, static_argnames=("interpret",))
def _tc_loss(inputs, twr, tb, tec, swr, sb, sec, interpret=False):
    out = pl.pallas_call(
        _tc_body,
        grid=(_GRID,),
        in_specs=[
            pl.BlockSpec((_TB, D), lambda i: (i, 0)),
            pl.BlockSpec((B, D), lambda i: (0, 0)),
            pl.BlockSpec((B, D), lambda i: (0, 0)),
            pl.BlockSpec((B, 1), lambda i: (0, 0)),
            pl.BlockSpec((B, 1), lambda i: (0, 0)),
            pl.BlockSpec((S, D), lambda i: (0, 0)),
            pl.BlockSpec((S, 1), lambda i: (0, 0)),
            pl.BlockSpec((S, 1), lambda i: (0, 0)),
        ],
        out_specs=pl.BlockSpec(memory_space=pltpu.SMEM),
        out_shape=jax.ShapeDtypeStruct((1, 1), jnp.float32),
        scratch_shapes=[pltpu.VMEM((S, 128), jnp.bfloat16)],
        interpret=interpret,
    )(inputs, inputs, twr, tb, tec, swr, sb, sec)
    return out[0, 0]


def kernel(inputs, labels, weights, biases, sampled_candidates,
           true_expected_count, sampled_expected_count):
    ids = jnp.concatenate(
        [labels.reshape(-1).astype(jnp.int32),
         sampled_candidates.astype(jnp.int32)], axis=0)
    ids3 = ids.reshape(_NW, _NCHUNK, _CHUNK)
    tw, tb, sw, sb = _sc_gather(ids3, weights, biases)
    return _tc_loss(inputs,
                    tw, tb.reshape(B, 1),
                    true_expected_count,
                    sw, sb.reshape(S, 1),
                    sampled_expected_count.reshape(S, 1))
